# Initial kernel scaffold; baseline (speedup 1.0000x reference)
import jax
import jax.numpy as jnp
from jax.experimental import pallas as pl


def kernel(poi_ids, cate_ids, lons, lats, id_table, cate_table):
    raise NotImplementedError("write your pallas kernel here")


# speedup vs baseline: 3.1444x; 3.1444x over previous
import functools
import math

import jax
import jax.numpy as jnp
from jax import lax
from jax.experimental import pallas as pl
from jax.experimental.pallas import tpu as pltpu
from jax.experimental.pallas import tpu_sc as plsc

POI_NUM = 1000000
CATE_NUM = 1000
EMBED_DIM = 64
FREQ_DIM = 16
LANE = 128
NROWS = 6400
NW = 32
RPT = NROWS // NW
CHR = 2

SCALE = 500.0 * math.pi
LON_MIN, LON_DEN = -180.0, 360.0
LAT_MIN, LAT_DEN = -90.0, 180.0
INV2PI = 1.0 / (2.0 * math.pi)
TWOPI = 2.0 * math.pi

S3, S5, S7, S9 = (-0.16664868466991684, 0.008317582019405747,
                  -0.00019381942439054465, 2.2017308075501785e-06)
C2, C4, C6, C8, C10 = (-0.4999967710450516, 0.04166173650283906,
                       -0.0013864431401809309, 2.4270696788300516e-05,
                       -2.2259710215355444e-07)


def _f32(x):
  return jnp.float32(x)


def _sincos(a):
  t = a * _f32(INV2PI) + _f32(0.5)
  n = t.astype(jnp.int32).astype(jnp.float32)
  r = a - n * _f32(TWOPI)
  x2 = r * r
  s = _f32(S9)
  s = s * x2 + _f32(S7)
  s = s * x2 + _f32(S5)
  s = s * x2 + _f32(S3)
  s = s * x2 + _f32(1.0)
  s = s * r
  c = _f32(C10)
  c = c * x2 + _f32(C8)
  c = c * x2 + _f32(C6)
  c = c * x2 + _f32(C4)
  c = c * x2 + _f32(C2)
  c = c * x2 + _f32(1.0)
  return s, c


def _body(poi_h, cid_h, lon_h, lat_h, idt_h, ct_h, out_h,
          ctv, idxv, cidv, lonv, latv, rows, sem):
  wid = lax.axis_index("s") * 2 + lax.axis_index("c")
  base = wid * RPT

  pltpu.sync_copy(ct_h, ctv)

  iota = lax.iota(jnp.int32, 16)
  freqs = jnp.exp(iota.astype(jnp.float32) * _f32(-(math.log(10000.0) / FREQ_DIM)))
  offs = [iota + jnp.int32(k * 16) for k in range(4)]

  def chunk(c, carry):
    r0 = base + c * CHR
    pltpu.sync_copy(poi_h.at[pl.ds(r0, CHR)], idxv)
    pltpu.sync_copy(cid_h.at[pl.ds(r0, CHR)], cidv)
    pltpu.sync_copy(lon_h.at[pl.ds(r0, CHR)], lonv)
    pltpu.sync_copy(lat_h.at[pl.ds(r0, CHR)], latv)
    cps = [pltpu.async_copy(idt_h.at[idxv.at[j]], rows.at[j], sem)
           for j in range(CHR)]
    for cp in cps:
      cp.wait()

    lon_c1 = _f32(SCALE / LON_DEN)
    lon_c2 = _f32(-LON_MIN / LON_DEN * SCALE)
    lat_c1 = _f32(SCALE / LAT_DEN)
    lat_c2 = _f32(-LAT_MIN / LAT_DEN * SCALE)
    for j in range(CHR):
      def grp(g, _, j=j):
        b16 = g * 16
        cidvec = cidv[j, pl.ds(b16, 16)]
        lonp = lonv[j, pl.ds(b16, 16)] * lon_c1 + lon_c2
        latp = latv[j, pl.ds(b16, 16)] * lat_c1 + lat_c2
        for e in range(16):
          sl, cl = _sincos(jnp.full((16,), lonp[e], jnp.float32) * freqs)
          st, ct = _sincos(jnp.full((16,), latp[e], jnp.float32) * freqs)
          cb = cidvec[e] * jnp.int32(EMBED_DIM)
          r = b16 + e
          for k, trig in enumerate((sl, cl, st, ct)):
            cate_k = ctv[pl.ds(cb + k * 16, 16)]
            cur = rows[j, r, pl.ds(k * 16, 16)]
            rows[j, r, pl.ds(k * 16, 16)] = cur + cate_k + trig
        return 0

      lax.fori_loop(0, LANE // 16, grp, 0)

    pltpu.sync_copy(rows, out_h.at[pl.ds(r0, CHR)])
    return carry

  lax.fori_loop(0, RPT // CHR, chunk, 0)


@jax.jit
def _embed(poi, cid, lon, lat, id_table, ct_flat):
  mesh = plsc.VectorSubcoreMesh(core_axis_name="c", subcore_axis_name="s")
  call = functools.partial(
      pl.kernel,
      out_type=jax.ShapeDtypeStruct((NROWS, LANE, EMBED_DIM), jnp.float32),
      mesh=mesh,
      scratch_types=[
          pltpu.VMEM((CATE_NUM * EMBED_DIM,), jnp.float32),
          pltpu.VMEM((CHR, LANE), jnp.int32),
          pltpu.VMEM((CHR, LANE), jnp.int32),
          pltpu.VMEM((CHR, LANE), jnp.float32),
          pltpu.VMEM((CHR, LANE), jnp.float32),
          pltpu.VMEM((CHR, LANE, EMBED_DIM), jnp.float32),
          pltpu.SemaphoreType.DMA,
      ],
      compiler_params=pltpu.CompilerParams(use_tc_tiling_on_sc=False),
  )(_body)
  return call(poi, cid, lon, lat, id_table, ct_flat)


def kernel(poi_ids, cate_ids, lons, lats, id_table, cate_table):
  poi = poi_ids.astype(jnp.int32).reshape(NROWS, LANE)
  cid = cate_ids.astype(jnp.int32).reshape(NROWS, LANE)
  lon = lons.reshape(NROWS, LANE)
  lat = lats.reshape(NROWS, LANE)
  out = _embed(poi, cid, lon, lat, id_table, cate_table.reshape(-1))
  return out.reshape(poi_ids.shape[0], poi_ids.shape[1], EMBED_DIM)


# speedup vs baseline: 3.5253x; 1.1211x over previous
import functools
import math

import jax
import jax.numpy as jnp
from jax import lax
from jax.experimental import pallas as pl
from jax.experimental.pallas import tpu as pltpu
from jax.experimental.pallas import tpu_sc as plsc

POI_NUM = 1000000
CATE_NUM = 1000
EMBED_DIM = 64
FREQ_DIM = 16
LANE = 128
NROWS = 6400
NW = 32
RPT = NROWS // NW
CHR = 2

SCALE = 500.0 * math.pi
LON_MIN, LON_DEN = -180.0, 360.0
LAT_MIN, LAT_DEN = -90.0, 180.0
INV2PI = 1.0 / (2.0 * math.pi)
TWOPI = 2.0 * math.pi

S3, S5, S7, S9 = (-0.16664868466991684, 0.008317582019405747,
                  -0.00019381942439054465, 2.2017308075501785e-06)
C2, C4, C6, C8, C10 = (-0.4999967710450516, 0.04166173650283906,
                       -0.0013864431401809309, 2.4270696788300516e-05,
                       -2.2259710215355444e-07)


def _f32(x):
  return jnp.float32(x)


def _sincos(a):
  t = a * _f32(INV2PI) + _f32(0.5)
  n = t.astype(jnp.int32).astype(jnp.float32)
  r = a - n * _f32(TWOPI)
  x2 = r * r
  s = _f32(S9)
  s = s * x2 + _f32(S7)
  s = s * x2 + _f32(S5)
  s = s * x2 + _f32(S3)
  s = s * x2 + _f32(1.0)
  s = s * r
  c = _f32(C10)
  c = c * x2 + _f32(C8)
  c = c * x2 + _f32(C6)
  c = c * x2 + _f32(C4)
  c = c * x2 + _f32(C2)
  c = c * x2 + _f32(1.0)
  return s, c


NCH = RPT // CHR


def _body(poi_h, cid_h, lon_h, lat_h, idt_h, ct_h, out_h,
          ctv, idxv0, idxv1, cidv0, cidv1, lonv0, lonv1, latv0, latv1,
          rows0, rows1, sin0, sin1, sg0, sg1, so0, so1):
  wid = lax.axis_index("s") * 2 + lax.axis_index("c")
  base = wid * RPT
  idxv = (idxv0, idxv1)
  cidv = (cidv0, cidv1)
  lonv = (lonv0, lonv1)
  latv = (latv0, latv1)
  rows = (rows0, rows1)
  sin_ = (sin0, sin1)
  sg = (sg0, sg1)
  so = (so0, so1)

  pltpu.sync_copy(ct_h, ctv)

  iota = lax.iota(jnp.int32, 16)
  freqs = jnp.exp(iota.astype(jnp.float32) * _f32(-(math.log(10000.0) / FREQ_DIM)))
  lon_c1 = _f32(SCALE / LON_DEN)
  lon_c2 = _f32(-LON_MIN / LON_DEN * SCALE)
  lat_c1 = _f32(SCALE / LAT_DEN)
  lat_c2 = _f32(-LAT_MIN / LAT_DEN * SCALE)

  def issue_inputs(c, b):
    r0 = base + c * CHR
    return [pltpu.async_copy(src.at[pl.ds(r0, CHR)], dst, sin_[b])
            for src, dst in ((poi_h, idxv[b]), (cid_h, cidv[b]),
                             (lon_h, lonv[b]), (lat_h, latv[b]))]

  def issue_gathers(b):
    for j in range(CHR):
      pltpu.async_copy(idt_h.at[idxv[b].at[j]], rows[b].at[j], sg[b])

  def wait_gathers(b):
    for j in range(CHR):
      pltpu.make_async_copy(idt_h.at[idxv[b].at[j]], rows[b].at[j],
                            sg[b]).wait()

  def wait_out(b):
    pltpu.make_async_copy(rows[b], out_h.at[pl.ds(base, CHR)], so[b]).wait()

  def compute(b):
    for j in range(CHR):
      def grp(g, _, j=j, b=b):
        b16 = g * 16
        cidvec = cidv[b][j, pl.ds(b16, 16)]
        lonp = lonv[b][j, pl.ds(b16, 16)] * lon_c1 + lon_c2
        latp = latv[b][j, pl.ds(b16, 16)] * lat_c1 + lat_c2
        for e in range(16):
          sl, cl = _sincos(jnp.full((16,), lonp[e], jnp.float32) * freqs)
          st, ct = _sincos(jnp.full((16,), latp[e], jnp.float32) * freqs)
          cb = cidvec[e] * jnp.int32(EMBED_DIM)
          r = b16 + e
          for k, trig in enumerate((sl, cl, st, ct)):
            cate_k = ctv[pl.ds(cb + k * 16, 16)]
            cur = rows[b][j, r, pl.ds(k * 16, 16)]
            rows[b][j, r, pl.ds(k * 16, 16)] = cur + cate_k + trig
        return 0

      lax.fori_loop(0, LANE // 16, grp, 0)

  for cp in issue_inputs(0, 0):
    cp.wait()
  issue_gathers(0)

  def step(s, carry):
    for b in range(2):
      c = s * 2 + b
      nxt = issue_inputs(c + 1, 1 - b) if b == 0 else None
      if b == 1:
        @pl.when(s < NCH // 2 - 1)
        def _():
          for cp in issue_inputs(c + 1, 1 - b):
            cp.wait()
      wait_gathers(b)
      compute(b)
      if b == 0:
        for cp in nxt:
          cp.wait()

        @pl.when(s > 0)
        def _():
          wait_out(1 - b)
        issue_gathers(1 - b)
      else:
        @pl.when(s < NCH // 2 - 1)
        def _():
          wait_out(1 - b)
          issue_gathers(1 - b)
      r0 = base + c * CHR
      pltpu.async_copy(rows[b], out_h.at[pl.ds(r0, CHR)], so[b])
    return carry

  lax.fori_loop(0, NCH // 2, step, 0)
  wait_out(0)
  wait_out(1)


@jax.jit
def _embed(poi, cid, lon, lat, id_table, ct_flat):
  mesh = plsc.VectorSubcoreMesh(core_axis_name="c", subcore_axis_name="s")
  call = functools.partial(
      pl.kernel,
      out_type=jax.ShapeDtypeStruct((NROWS, LANE, EMBED_DIM), jnp.float32),
      mesh=mesh,
      scratch_types=(
          [pltpu.VMEM((CATE_NUM * EMBED_DIM,), jnp.float32)]
          + [pltpu.VMEM((CHR, LANE), jnp.int32)] * 4
          + [pltpu.VMEM((CHR, LANE), jnp.float32)] * 4
          + [pltpu.VMEM((CHR, LANE, EMBED_DIM), jnp.float32)] * 2
          + [pltpu.SemaphoreType.DMA] * 6
      ),
      compiler_params=pltpu.CompilerParams(use_tc_tiling_on_sc=False),
  )(_body)
  return call(poi, cid, lon, lat, id_table, ct_flat)


def kernel(poi_ids, cate_ids, lons, lats, id_table, cate_table):
  poi = poi_ids.astype(jnp.int32).reshape(NROWS, LANE)
  cid = cate_ids.astype(jnp.int32).reshape(NROWS, LANE)
  lon = lons.reshape(NROWS, LANE)
  lat = lats.reshape(NROWS, LANE)
  out = _embed(poi, cid, lon, lat, id_table, cate_table.reshape(-1))
  return out.reshape(poi_ids.shape[0], poi_ids.shape[1], EMBED_DIM)
